# baseline (speedup 1.0000x reference)
import jax
import jax.numpy as jnp
from jax import lax
from jax.experimental import pallas as pl
from jax.experimental.pallas import tpu as pltpu
from jax.experimental.pallas import tpu_sc as plsc

N_NODES = 10000
N_EDGES = 320000
D_FEAT = 128
D_HID = 16
N_CLASSES = 10

NC, NS = 2, 16
NW = NC * NS
CHUNK = 128
NBUF = 8
GAHEAD = 4
K0 = 104
K1 = 56
CH_TOT = NS * (K0 + K1)
CH_PAD = NS * K0 + NS * K1 + (K0 - K1)
E_PAD = CH_PAD * CHUNK

ACC_ROWS_PER_SUBCORE = 632
ACC_ROWS = NS * ACC_ROWS_PER_SUBCORE
FROWS = ACC_ROWS * D_HID // 128
FROWS_REAL = N_NODES * D_HID // 128
XF_COLS = 8 * D_FEAT

_mesh = plsc.VectorSubcoreMesh(core_axis_name="c", subcore_axis_name="s")
_f32 = jnp.float32


def _edge_pass_kernel(g_hbm, src_hbm, dst_hbm, zeros_hbm, out_hbm,
                      acc_sh, g_sh, src_v, dst_v,
                      r0, r1, r2, r3, r4, r5, r6, r7,
                      g0, g1, g2, g3, g4, g5, g6, g7,
                      t0, t1, t2, t3, t4, t5, t6, t7):
    rows = (r0, r1, r2, r3, r4, r5, r6, r7)
    gsems = (g0, g1, g2, g3, g4, g5, g6, g7)
    ssems = (t0, t1, t2, t3, t4, t5, t6, t7)
    c = lax.axis_index("c")
    s = lax.axis_index("s")
    kc = jnp.where(c == 0, K0, K1)
    base = s * K0 + c * (NS * K0) + c * s * (K1 - K0)
    row0 = s * ACC_ROWS_PER_SUBCORE
    pltpu.sync_copy(zeros_hbm.at[pl.ds(row0, ACC_ROWS_PER_SUBCORE)],
                    acc_sh.at[pl.ds(row0, ACC_ROWS_PER_SUBCORE)])
    pltpu.sync_copy(g_hbm.at[pl.ds(row0, ACC_ROWS_PER_SUBCORE)],
                    g_sh.at[pl.ds(row0, ACC_ROWS_PER_SUBCORE)])
    pltpu.sync_copy(src_hbm.at[pl.ds(base, K0)], src_v)
    pltpu.sync_copy(dst_hbm.at[pl.ds(base, K0)], dst_v)
    plsc.subcore_barrier()

    def gather_start(j, b):
        pltpu.async_copy(g_sh.at[src_v.at[j]], rows[b], gsems[b])

    def gather_wait(j, b):
        pltpu.make_async_copy(g_sh.at[src_v.at[j]], rows[b], gsems[b]).wait()

    def scatter_start(j, b):
        pltpu.async_copy(rows[b], acc_sh.at[dst_v.at[j]], ssems[b], add=True)

    def scatter_wait(j, b):
        pltpu.make_async_copy(rows[b], acc_sh.at[dst_v.at[j]],
                              ssems[b]).wait()

    for b in range(GAHEAD):
        gather_start(b, b)
    for b in range(NBUF):
        j = b
        gather_wait(j, b)
        scatter_start(j, b)
        if j >= GAHEAD:
            scatter_wait(j - GAHEAD, (j - GAHEAD) % NBUF)
        gather_start(j + GAHEAD, (j + GAHEAD) % NBUF)

    @pl.loop(1, kc // NBUF - 1)
    def _(g):
        gbase = g * NBUF
        for b in range(NBUF):
            j = gbase + b
            gather_wait(j, b)
            scatter_start(j, b)
            scatter_wait(j - GAHEAD, (b - GAHEAD) % NBUF)
            gather_start(j + GAHEAD, (b + GAHEAD) % NBUF)

    tail = kc - NBUF
    for b in range(NBUF):
        j = tail + b
        gather_wait(j, b)
        scatter_start(j, b)
        scatter_wait(j - GAHEAD, (b - GAHEAD) % NBUF)
        if b + GAHEAD < NBUF:
            gather_start(j + GAHEAD, (b + GAHEAD) % NBUF)
    for b in range(GAHEAD):
        j = kc - GAHEAD + b
        scatter_wait(j, (b + GAHEAD) % NBUF)

    plsc.subcore_barrier()
    pltpu.sync_copy(acc_sh.at[pl.ds(row0, ACC_ROWS_PER_SUBCORE)],
                    out_hbm.at[c, pl.ds(row0, ACC_ROWS_PER_SUBCORE)])


_edge_pass = pl.kernel(
    _edge_pass_kernel,
    out_type=jax.ShapeDtypeStruct((NC, ACC_ROWS, D_HID), _f32),
    mesh=_mesh,
    scratch_types=(
        [pltpu.VMEM_SHARED((ACC_ROWS, D_HID), _f32),
         pltpu.VMEM_SHARED((ACC_ROWS, D_HID), _f32),
         pltpu.VMEM((K0, CHUNK), jnp.int32),
         pltpu.VMEM((K0, CHUNK), jnp.int32)]
        + [pltpu.VMEM((CHUNK, D_HID), _f32)] * NBUF
        + [pltpu.SemaphoreType.DMA] * (2 * NBUF)
    ),
    compiler_params=pltpu.CompilerParams(use_tc_tiling_on_sc=False),
)


def _deg_pass_kernel(dst_hbm, zeros_hbm, ones_hbm, out_hbm,
                     acc_sh, dst_v, ones_v, sem):
    c = lax.axis_index("c")
    s = lax.axis_index("s")
    kc = jnp.where(c == 0, K0, K1)
    base0 = s * K0 + c * (NS * K0) + c * s * (K1 - K0)
    row0 = s * ACC_ROWS_PER_SUBCORE
    pltpu.sync_copy(zeros_hbm.at[pl.ds(row0, ACC_ROWS_PER_SUBCORE)],
                    acc_sh.at[pl.ds(row0, ACC_ROWS_PER_SUBCORE)])
    pltpu.sync_copy(dst_hbm.at[pl.ds(base0, K0)], dst_v)
    pltpu.sync_copy(ones_hbm, ones_v)
    plsc.subcore_barrier()

    @pl.loop(0, kc // 8)
    def _(g):
        gbase = g * 8
        for b in range(8):
            pltpu.async_copy(ones_v, acc_sh.at[dst_v.at[gbase + b]], sem,
                             add=True)
        for b in range(8):
            pltpu.make_async_copy(ones_v, acc_sh.at[dst_v.at[gbase + b]],
                                  sem).wait()

    plsc.subcore_barrier()
    pltpu.sync_copy(acc_sh.at[pl.ds(row0, ACC_ROWS_PER_SUBCORE)],
                    out_hbm.at[c, pl.ds(row0, ACC_ROWS_PER_SUBCORE)])


_deg_pass = pl.kernel(
    _deg_pass_kernel,
    out_type=jax.ShapeDtypeStruct((NC, ACC_ROWS, D_HID), _f32),
    mesh=_mesh,
    scratch_types=[
        pltpu.VMEM_SHARED((ACC_ROWS, D_HID), _f32),
        pltpu.VMEM((K0, CHUNK), jnp.int32),
        pltpu.VMEM((CHUNK, D_HID), _f32),
        pltpu.SemaphoreType.DMA,
    ],
    compiler_params=pltpu.CompilerParams(use_tc_tiling_on_sc=False),
)


def _mm1_body(xf_ref, w1bd_ref, hf_ref):
    hf_ref[...] = jnp.dot(xf_ref[...], w1bd_ref[...],
                          preferred_element_type=_f32)


def _prep2_body(degpf_ref, h1f_ref, dinvf_ref, g1f_ref):
    deg = degpf_ref[0] + degpf_ref[1] + 1.0
    dinvf = lax.rsqrt(deg)
    dinvf_ref[...] = dinvf
    g1f_ref[...] = jnp.concatenate(
        [dinvf[:FROWS_REAL] * h1f_ref[...],
         jnp.zeros((FROWS - FROWS_REAL, 128), _f32)], axis=0)


def _mid_body(accpf_ref, g1f_ref, dinvf_ref, b1f_ref, w2bd_ref, g2f_ref):
    agg = accpf_ref[0] + accpf_ref[1] + g1f_ref[...]
    h = jnp.maximum(dinvf_ref[...] * agg + b1f_ref[...], 0.0)
    h2 = jnp.dot(h, w2bd_ref[...], preferred_element_type=_f32,
                 precision=lax.Precision.HIGHEST)
    g2f_ref[...] = dinvf_ref[...] * h2


def _final_body(accpf_ref, g2f_ref, dinvf_ref, b2f_ref, of_ref):
    agg = accpf_ref[0, :FROWS_REAL] + accpf_ref[1, :FROWS_REAL] \
        + g2f_ref[0:FROWS_REAL]
    of_ref[...] = dinvf_ref[0:FROWS_REAL] * agg + b2f_ref[...]


def kernel(x, edge_index, W1, b1, W2, b2):
    flat = edge_index.astype(jnp.int32).reshape(2 * N_EDGES)
    flat = lax.optimization_barrier(flat)
    both = jnp.concatenate(
        [flat[:N_EDGES], jnp.zeros((E_PAD - N_EDGES,), jnp.int32),
         flat[N_EDGES:], jnp.full((E_PAD - N_EDGES,), N_NODES, jnp.int32)])
    src = both[:E_PAD].reshape(CH_PAD, CHUNK)
    dst = both[E_PAD:].reshape(CH_PAD, CHUNK)
    zeros = jnp.zeros((ACC_ROWS, D_HID), _f32)
    ones = jnp.ones((CHUNK, D_HID), _f32)
    eye8 = jnp.eye(8, dtype=_f32)
    w1bd = jnp.kron(eye8, W1)
    W2p = jnp.pad(W2, ((0, 0), (0, D_HID - N_CLASSES)))
    w2bd = jnp.kron(eye8, W2p)
    b1f = jnp.tile(b1, 8).reshape(1, 128)
    b2f = jnp.tile(jnp.pad(b2, (0, D_HID - N_CLASSES)), 8).reshape(1, 128)
    xf = x.reshape(FROWS_REAL, XF_COLS)

    degp = _deg_pass(dst, zeros, ones)
    degpf = degp.reshape(NC, FROWS, 128)
    h1f = pl.pallas_call(
        _mm1_body,
        out_shape=jax.ShapeDtypeStruct((FROWS_REAL, 128), _f32),
    )(xf, w1bd)
    dinvf, g1f = pl.pallas_call(
        _prep2_body,
        out_shape=(jax.ShapeDtypeStruct((FROWS, 128), _f32),
                   jax.ShapeDtypeStruct((FROWS, 128), _f32)),
    )(degpf, h1f)
    acc1 = _edge_pass(g1f.reshape(ACC_ROWS, D_HID), src, dst, zeros)
    g2f = pl.pallas_call(
        _mid_body,
        out_shape=jax.ShapeDtypeStruct((FROWS, 128), _f32),
    )(acc1.reshape(NC, FROWS, 128), g1f, dinvf, b1f, w2bd)
    acc2 = _edge_pass(g2f.reshape(ACC_ROWS, D_HID), src, dst, zeros)
    resf = pl.pallas_call(
        _final_body,
        out_shape=jax.ShapeDtypeStruct((FROWS_REAL, 128), _f32),
    )(acc2.reshape(NC, FROWS, 128), g2f, dinvf, b2f)
    return resf.reshape(N_NODES, D_HID)[:, :N_CLASSES]


# speedup vs baseline: 1.0237x; 1.0237x over previous
import jax
import jax.numpy as jnp
from jax import lax
from jax.experimental import pallas as pl
from jax.experimental.pallas import tpu as pltpu
from jax.experimental.pallas import tpu_sc as plsc

N_NODES = 10000
N_EDGES = 320000
D_FEAT = 128
D_HID = 16
N_CLASSES = 10

NC, NS = 2, 16
NW = NC * NS
CHUNK = 128
NBUF = 8
GAHEAD = 4
K0 = 96
K1 = 64
CH_TOT = NS * (K0 + K1)
CH_PAD = NS * K0 + NS * K1 + (K0 - K1)
E_PAD = CH_PAD * CHUNK

ACC_ROWS_PER_SUBCORE = 632
ACC_ROWS = NS * ACC_ROWS_PER_SUBCORE
FROWS = ACC_ROWS * D_HID // 128
FROWS_REAL = N_NODES * D_HID // 128
XF_COLS = 8 * D_FEAT

_mesh = plsc.VectorSubcoreMesh(core_axis_name="c", subcore_axis_name="s")
_f32 = jnp.float32


def _edge_pass_kernel(g_hbm, src_hbm, dst_hbm, zeros_hbm, out_hbm,
                      acc_sh, g_sh, src_v, dst_v,
                      r0, r1, r2, r3, r4, r5, r6, r7,
                      g0, g1, g2, g3, g4, g5, g6, g7,
                      t0, t1, t2, t3, t4, t5, t6, t7):
    rows = (r0, r1, r2, r3, r4, r5, r6, r7)
    gsems = (g0, g1, g2, g3, g4, g5, g6, g7)
    ssems = (t0, t1, t2, t3, t4, t5, t6, t7)
    c = lax.axis_index("c")
    s = lax.axis_index("s")
    kc = jnp.where(c == 0, K0, K1)
    base = s * K0 + c * (NS * K0) + c * s * (K1 - K0)
    row0 = s * ACC_ROWS_PER_SUBCORE
    pltpu.sync_copy(zeros_hbm.at[pl.ds(row0, ACC_ROWS_PER_SUBCORE)],
                    acc_sh.at[pl.ds(row0, ACC_ROWS_PER_SUBCORE)])
    pltpu.sync_copy(g_hbm.at[pl.ds(row0, ACC_ROWS_PER_SUBCORE)],
                    g_sh.at[pl.ds(row0, ACC_ROWS_PER_SUBCORE)])
    pltpu.sync_copy(src_hbm.at[pl.ds(base, K0)], src_v)
    pltpu.sync_copy(dst_hbm.at[pl.ds(base, K0)], dst_v)
    plsc.subcore_barrier()

    def gather_start(j, b):
        pltpu.async_copy(g_sh.at[src_v.at[j]], rows[b], gsems[b])

    def gather_wait(j, b):
        pltpu.make_async_copy(g_sh.at[src_v.at[j]], rows[b], gsems[b]).wait()

    def scatter_start(j, b):
        pltpu.async_copy(rows[b], acc_sh.at[dst_v.at[j]], ssems[b], add=True)

    def scatter_wait(j, b):
        pltpu.make_async_copy(rows[b], acc_sh.at[dst_v.at[j]],
                              ssems[b]).wait()

    for b in range(GAHEAD):
        gather_start(b, b)
    for b in range(NBUF):
        j = b
        gather_wait(j, b)
        scatter_start(j, b)
        if j >= GAHEAD:
            scatter_wait(j - GAHEAD, (j - GAHEAD) % NBUF)
        gather_start(j + GAHEAD, (j + GAHEAD) % NBUF)

    @pl.loop(1, kc // NBUF - 1)
    def _(g):
        gbase = g * NBUF
        for b in range(NBUF):
            j = gbase + b
            gather_wait(j, b)
            scatter_start(j, b)
            scatter_wait(j - GAHEAD, (b - GAHEAD) % NBUF)
            gather_start(j + GAHEAD, (b + GAHEAD) % NBUF)

    tail = kc - NBUF
    for b in range(NBUF):
        j = tail + b
        gather_wait(j, b)
        scatter_start(j, b)
        scatter_wait(j - GAHEAD, (b - GAHEAD) % NBUF)
        if b + GAHEAD < NBUF:
            gather_start(j + GAHEAD, (b + GAHEAD) % NBUF)
    for b in range(GAHEAD):
        j = kc - GAHEAD + b
        scatter_wait(j, (b + GAHEAD) % NBUF)

    plsc.subcore_barrier()
    pltpu.sync_copy(acc_sh.at[pl.ds(row0, ACC_ROWS_PER_SUBCORE)],
                    out_hbm.at[c, pl.ds(row0, ACC_ROWS_PER_SUBCORE)])


_edge_pass = pl.kernel(
    _edge_pass_kernel,
    out_type=jax.ShapeDtypeStruct((NC, ACC_ROWS, D_HID), _f32),
    mesh=_mesh,
    scratch_types=(
        [pltpu.VMEM_SHARED((ACC_ROWS, D_HID), _f32),
         pltpu.VMEM_SHARED((ACC_ROWS, D_HID), _f32),
         pltpu.VMEM((K0, CHUNK), jnp.int32),
         pltpu.VMEM((K0, CHUNK), jnp.int32)]
        + [pltpu.VMEM((CHUNK, D_HID), _f32)] * NBUF
        + [pltpu.SemaphoreType.DMA] * (2 * NBUF)
    ),
    compiler_params=pltpu.CompilerParams(use_tc_tiling_on_sc=False),
)


def _deg_pass_kernel(dst_hbm, zeros_hbm, ones_hbm, out_hbm,
                     acc_sh, dst_v, ones_v, sem):
    c = lax.axis_index("c")
    s = lax.axis_index("s")
    kc = jnp.where(c == 0, K0, K1)
    base0 = s * K0 + c * (NS * K0) + c * s * (K1 - K0)
    row0 = s * ACC_ROWS_PER_SUBCORE
    pltpu.sync_copy(zeros_hbm.at[pl.ds(row0, ACC_ROWS_PER_SUBCORE)],
                    acc_sh.at[pl.ds(row0, ACC_ROWS_PER_SUBCORE)])
    pltpu.sync_copy(dst_hbm.at[pl.ds(base0, K0)], dst_v)
    pltpu.sync_copy(ones_hbm, ones_v)
    plsc.subcore_barrier()

    @pl.loop(0, kc // 8)
    def _(g):
        gbase = g * 8
        for b in range(8):
            pltpu.async_copy(ones_v, acc_sh.at[dst_v.at[gbase + b]], sem,
                             add=True)
        for b in range(8):
            pltpu.make_async_copy(ones_v, acc_sh.at[dst_v.at[gbase + b]],
                                  sem).wait()

    plsc.subcore_barrier()
    pltpu.sync_copy(acc_sh.at[pl.ds(row0, ACC_ROWS_PER_SUBCORE)],
                    out_hbm.at[c, pl.ds(row0, ACC_ROWS_PER_SUBCORE)])


_deg_pass = pl.kernel(
    _deg_pass_kernel,
    out_type=jax.ShapeDtypeStruct((NC, ACC_ROWS, D_HID), _f32),
    mesh=_mesh,
    scratch_types=[
        pltpu.VMEM_SHARED((ACC_ROWS, D_HID), _f32),
        pltpu.VMEM((K0, CHUNK), jnp.int32),
        pltpu.VMEM((CHUNK, D_HID), _f32),
        pltpu.SemaphoreType.DMA,
    ],
    compiler_params=pltpu.CompilerParams(use_tc_tiling_on_sc=False),
)


def _mm1_body(xf_ref, w1bd_ref, hf_ref):
    hf_ref[...] = jnp.dot(xf_ref[...], w1bd_ref[...],
                          preferred_element_type=_f32)


def _prep2_body(degpf_ref, h1f_ref, dinvf_ref, g1f_ref):
    deg = degpf_ref[0] + degpf_ref[1] + 1.0
    dinvf = lax.rsqrt(deg)
    dinvf_ref[...] = dinvf
    g1f_ref[...] = jnp.concatenate(
        [dinvf[:FROWS_REAL] * h1f_ref[...],
         jnp.zeros((FROWS - FROWS_REAL, 128), _f32)], axis=0)


def _mid_body(accpf_ref, g1f_ref, dinvf_ref, b1f_ref, w2bd_ref, g2f_ref):
    agg = accpf_ref[0] + accpf_ref[1] + g1f_ref[...]
    h = jnp.maximum(dinvf_ref[...] * agg + b1f_ref[...], 0.0)
    h2 = jnp.dot(h, w2bd_ref[...], preferred_element_type=_f32,
                 precision=lax.Precision.HIGHEST)
    g2f_ref[...] = dinvf_ref[...] * h2


def _final_body(accpf_ref, g2f_ref, dinvf_ref, b2f_ref, of_ref):
    agg = accpf_ref[0, :FROWS_REAL] + accpf_ref[1, :FROWS_REAL] \
        + g2f_ref[0:FROWS_REAL]
    of_ref[...] = dinvf_ref[0:FROWS_REAL] * agg + b2f_ref[...]


def kernel(x, edge_index, W1, b1, W2, b2):
    flat = edge_index.astype(jnp.int32).reshape(2 * N_EDGES)
    flat = lax.optimization_barrier(flat)
    both = jnp.concatenate(
        [flat[:N_EDGES], jnp.zeros((E_PAD - N_EDGES,), jnp.int32),
         flat[N_EDGES:], jnp.full((E_PAD - N_EDGES,), N_NODES, jnp.int32)])
    src = both[:E_PAD].reshape(CH_PAD, CHUNK)
    dst = both[E_PAD:].reshape(CH_PAD, CHUNK)
    zeros = jnp.zeros((ACC_ROWS, D_HID), _f32)
    ones = jnp.ones((CHUNK, D_HID), _f32)
    eye8 = jnp.eye(8, dtype=_f32)
    w1bd = jnp.kron(eye8, W1)
    W2p = jnp.pad(W2, ((0, 0), (0, D_HID - N_CLASSES)))
    w2bd = jnp.kron(eye8, W2p)
    b1f = jnp.tile(b1, 8).reshape(1, 128)
    b2f = jnp.tile(jnp.pad(b2, (0, D_HID - N_CLASSES)), 8).reshape(1, 128)
    xf = x.reshape(FROWS_REAL, XF_COLS)

    degp = _deg_pass(dst, zeros, ones)
    degpf = degp.reshape(NC, FROWS, 128)
    h1f = pl.pallas_call(
        _mm1_body,
        out_shape=jax.ShapeDtypeStruct((FROWS_REAL, 128), _f32),
    )(xf, w1bd)
    dinvf, g1f = pl.pallas_call(
        _prep2_body,
        out_shape=(jax.ShapeDtypeStruct((FROWS, 128), _f32),
                   jax.ShapeDtypeStruct((FROWS, 128), _f32)),
    )(degpf, h1f)
    acc1 = _edge_pass(g1f.reshape(ACC_ROWS, D_HID), src, dst, zeros)
    g2f = pl.pallas_call(
        _mid_body,
        out_shape=jax.ShapeDtypeStruct((FROWS, 128), _f32),
    )(acc1.reshape(NC, FROWS, 128), g1f, dinvf, b1f, w2bd)
    acc2 = _edge_pass(g2f.reshape(ACC_ROWS, D_HID), src, dst, zeros)
    resf = pl.pallas_call(
        _final_body,
        out_shape=jax.ShapeDtypeStruct((FROWS_REAL, 128), _f32),
    )(acc2.reshape(NC, FROWS, 128), g2f, dinvf, b2f)
    return resf.reshape(N_NODES, D_HID)[:, :N_CLASSES]
